# baseline (speedup 1.0000x reference)
import jax
import jax.numpy as jnp
from jax.experimental import pallas as pl

_BN = 2048
_KC = 4


def _heads_kernel(*refs):
    x_refs = refs[:_KC]
    (wc_ref, bc_ref, wco_ref, bco_ref, wm_ref, bm_ref,
     ws_ref, bs_ref, wb_ref, bb_ref) = refs[_KC:_KC + 10]
    scores_ref, color_ref, material_ref, state_ref, bbox_ref = refs[_KC + 10:]

    xs = [r[...].astype(jnp.bfloat16) for r in x_refs]
    chunk = xs[0].shape[1]
    dims = (((1,), (1,)), ((), ()))

    def head(w_ref, b_ref):
        acc = None
        for k, xk in enumerate(xs):
            y = jax.lax.dot_general(w_ref[:, k * chunk:(k + 1) * chunk], xk,
                                    dims, preferred_element_type=jnp.float32)
            acc = y if acc is None else acc + y
        return acc + b_ref[...]

    scores_ref[...] = head(wc_ref, bc_ref)
    color_ref[...] = head(wco_ref, bco_ref)
    material_ref[...] = head(wm_ref, bm_ref)
    state_ref[...] = head(ws_ref, bs_ref)
    bbox_ref[...] = head(wb_ref, bb_ref)


def kernel(x, W_cls, b_cls, W_color, b_color, W_material, b_material,
           W_state, b_state, W_bbox, b_bbox):
    n, c = x.shape
    chunk = c // _KC
    heads = [(W_cls, b_cls), (W_color, b_color), (W_material, b_material),
             (W_state, b_state), (W_bbox, b_bbox)]
    grid = (pl.cdiv(n, _BN),)

    in_specs = [pl.BlockSpec((_BN, chunk), lambda i, k=k: (i, k))
                for k in range(_KC)]
    operands = [x] * _KC
    full = pl.BlockSpec(None, lambda i: (0, 0))
    for W, b in heads:
        in_specs += [full, full]
        operands += [W.T.astype(jnp.bfloat16), b.reshape(-1, 1)]

    out_shapes = tuple(jax.ShapeDtypeStruct((W.shape[1], n), jnp.float32)
                       for W, _ in heads)
    out_specs = tuple(pl.BlockSpec((W.shape[1], _BN), lambda i: (0, i))
                      for W, _ in heads)

    outs = pl.pallas_call(
        _heads_kernel,
        grid=grid,
        in_specs=in_specs,
        out_specs=out_specs,
        out_shape=out_shapes,
    )(*operands)
    return tuple(jnp.transpose(o) for o in outs)


# speedup vs baseline: 1.4215x; 1.4215x over previous
import jax
import jax.numpy as jnp
from jax.experimental import pallas as pl

_BN = 2048
_DIMS = (91, 12, 10, 8, 364)
_OFFS = (0, 96, 112, 128, 136)
_TOTAL = 500


def _heads_kernel(x_ref, w_ref, b_ref,
                  scores_ref, color_ref, material_ref, state_ref, bbox_ref):
    x = x_ref[...].astype(jnp.bfloat16)
    y = jax.lax.dot_general(w_ref[...], x, (((1,), (1,)), ((), ())),
                            preferred_element_type=jnp.float32)
    y = y + b_ref[...]
    for ref, d, off in zip(
            (scores_ref, color_ref, material_ref, state_ref, bbox_ref),
            _DIMS, _OFFS):
        ref[...] = y[off:off + d]


def kernel(x, W_cls, b_cls, W_color, b_color, W_material, b_material,
           W_state, b_state, W_bbox, b_bbox):
    n, c = x.shape
    heads = [(W_cls, b_cls), (W_color, b_color), (W_material, b_material),
             (W_state, b_state), (W_bbox, b_bbox)]

    w_parts, b_parts = [], []
    for (W, b), d, off, nxt in zip(heads, _DIMS, _OFFS, _OFFS[1:] + (_TOTAL,)):
        pad = nxt - off - d
        w_parts.append(W.T.astype(jnp.bfloat16))
        b_parts.append(b.reshape(-1, 1))
        if pad:
            w_parts.append(jnp.zeros((pad, c), jnp.bfloat16))
            b_parts.append(jnp.zeros((pad, 1), jnp.float32))
    w_all = jnp.concatenate(w_parts, axis=0)
    b_all = jnp.concatenate(b_parts, axis=0)

    grid = (pl.cdiv(n, _BN),)
    in_specs = [pl.BlockSpec((_BN, c), lambda i: (i, 0)),
                pl.BlockSpec(None, lambda i: (0, 0)),
                pl.BlockSpec(None, lambda i: (0, 0))]
    out_shapes = tuple(jax.ShapeDtypeStruct((d, n), jnp.float32)
                       for d in _DIMS)
    out_specs = tuple(pl.BlockSpec((d, _BN), lambda i: (0, i))
                      for d in _DIMS)

    outs = pl.pallas_call(
        _heads_kernel,
        grid=grid,
        in_specs=in_specs,
        out_specs=out_specs,
        out_shape=out_shapes,
    )(x, w_all, b_all)
    return tuple(jnp.transpose(o) for o in outs)


# speedup vs baseline: 1.4327x; 1.0079x over previous
import jax
import jax.numpy as jnp
from jax.experimental import pallas as pl

_BN = 2048
_DIMS = (91, 12, 10, 8, 364)
_OFFS = (0, 96, 112, 128, 136)
_TOTAL = 500


def _heads_kernel(x_ref, w_ref, b_ref,
                  scores_ref, color_ref, material_ref, state_ref, bbox_ref):
    x = x_ref[...].astype(jnp.bfloat16)
    y = jax.lax.dot_general(w_ref[...], x, (((1,), (1,)), ((), ())),
                            preferred_element_type=jnp.float32)
    y = y + b_ref[...]
    for ref, d, off in zip(
            (scores_ref, color_ref, material_ref, state_ref, bbox_ref),
            _DIMS, _OFFS):
        ref[...] = y[off:off + d]


def kernel(x, W_cls, b_cls, W_color, b_color, W_material, b_material,
           W_state, b_state, W_bbox, b_bbox):
    n, c = x.shape
    heads = [(W_cls, b_cls), (W_color, b_color), (W_material, b_material),
             (W_state, b_state), (W_bbox, b_bbox)]

    w_parts, b_parts = [], []
    for (W, b), d, off, nxt in zip(heads, _DIMS, _OFFS, _OFFS[1:] + (_TOTAL,)):
        pad = nxt - off - d
        w_parts.append(W)
        b_parts.append(b)
        if pad:
            w_parts.append(jnp.zeros((c, pad), jnp.float32))
            b_parts.append(jnp.zeros((pad,), jnp.float32))
    w_all = jnp.concatenate(w_parts, axis=1).T.astype(jnp.bfloat16)
    b_all = jnp.concatenate(b_parts).reshape(-1, 1)

    grid = (pl.cdiv(n, _BN),)
    in_specs = [pl.BlockSpec((_BN, c), lambda i: (i, 0)),
                pl.BlockSpec(None, lambda i: (0, 0)),
                pl.BlockSpec(None, lambda i: (0, 0))]
    out_shapes = tuple(jax.ShapeDtypeStruct((d, n), jnp.float32)
                       for d in _DIMS)
    out_specs = tuple(pl.BlockSpec((d, _BN), lambda i: (0, i))
                      for d in _DIMS)

    outs = pl.pallas_call(
        _heads_kernel,
        grid=grid,
        in_specs=in_specs,
        out_specs=out_specs,
        out_shape=out_shapes,
    )(x, w_all, b_all)
    return tuple(jnp.transpose(o) for o in outs)


# speedup vs baseline: 1.8477x; 1.2896x over previous
import jax
import jax.numpy as jnp
from jax.experimental import pallas as pl
from jax.experimental.pallas import tpu as pltpu

_BN = 2048
_DIMS = (91, 12, 10, 8, 364)
_OFFS = (0, 96, 112, 128, 136)
_TOTAL = 500


def _heads_kernel(x_ref,
                  wc_ref, bc_ref, wco_ref, bco_ref, wm_ref, bm_ref,
                  ws_ref, bs_ref, wb_ref, bb_ref,
                  scores_ref, color_ref, material_ref, state_ref, bbox_ref,
                  w_s, b_s):
    w_refs = (wc_ref, wco_ref, wm_ref, ws_ref, wb_ref)
    b_refs = (bc_ref, bco_ref, bm_ref, bs_ref, bb_ref)

    @pl.when(pl.program_id(0) == 0)
    def _stack():
        for w_ref, b_ref, d, off in zip(w_refs, b_refs, _DIMS, _OFFS):
            w_s[off:off + d] = w_ref[...].astype(jnp.bfloat16)
            b_s[off:off + d] = jnp.swapaxes(b_ref[...], 0, 1)

    x = x_ref[...].astype(jnp.bfloat16)
    y = jax.lax.dot_general(w_s[...], x, (((1,), (1,)), ((), ())),
                            preferred_element_type=jnp.float32)
    y = y + b_s[...]
    for ref, d, off in zip(
            (scores_ref, color_ref, material_ref, state_ref, bbox_ref),
            _DIMS, _OFFS):
        ref[...] = y[off:off + d]


def kernel(x, W_cls, b_cls, W_color, b_color, W_material, b_material,
           W_state, b_state, W_bbox, b_bbox):
    n, c = x.shape
    heads = [(W_cls, b_cls), (W_color, b_color), (W_material, b_material),
             (W_state, b_state), (W_bbox, b_bbox)]

    grid = (pl.cdiv(n, _BN),)
    full = pl.BlockSpec(None, lambda i: (0, 0))
    in_specs = [pl.BlockSpec((_BN, c), lambda i: (i, 0))]
    operands = [x]
    for W, b in heads:
        in_specs += [full, full]
        operands += [W.T, b.reshape(1, -1)]

    out_shapes = tuple(jax.ShapeDtypeStruct((d, n), jnp.float32)
                       for d in _DIMS)
    out_specs = tuple(pl.BlockSpec((d, _BN), lambda i: (0, i))
                      for d in _DIMS)

    outs = pl.pallas_call(
        _heads_kernel,
        grid=grid,
        in_specs=in_specs,
        out_specs=out_specs,
        out_shape=out_shapes,
        scratch_shapes=[pltpu.VMEM((_TOTAL, c), jnp.bfloat16),
                        pltpu.VMEM((_TOTAL, 1), jnp.float32)],
    )(*operands)
    return tuple(jnp.transpose(o) for o in outs)
